# baseline (speedup 1.0000x reference)
import functools

import jax
import jax.numpy as jnp
from jax import lax
from jax.experimental import pallas as pl
from jax.experimental.pallas import tpu as pltpu
from jax.experimental.pallas import tpu_sc as plsc

NUM_USERS = 100000
EMBED_DIM = 128
BATCH = 16384

_NC = 2
_NS = 16
_NW = _NC * _NS
_B_PER_W = BATCH // _NW
_CHUNK = 64
_N_CHUNKS = _B_PER_W // _CHUNK


def _gather_call(idx, table):
  mesh = plsc.VectorSubcoreMesh(core_axis_name="c", subcore_axis_name="s")

  @functools.partial(
      pl.kernel,
      mesh=mesh,
      out_type=jax.ShapeDtypeStruct((BATCH, EMBED_DIM), jnp.float32),
      scratch_types=[
          pltpu.VMEM((_N_CHUNKS, _CHUNK), jnp.int32),
          pltpu.VMEM((_B_PER_W, EMBED_DIM), jnp.float32),
          *([pltpu.SemaphoreType.DMA] * _N_CHUNKS),
          pltpu.SemaphoreType.DMA,
      ],
  )
  def k(idx_hbm, table_hbm, out_hbm, idx_v, rows_v, *sems):
    gather_sems, out_sem = sems[:_N_CHUNKS], sems[_N_CHUNKS]
    wid = lax.axis_index("s") * _NC + lax.axis_index("c")
    base = wid * _B_PER_W
    pltpu.sync_copy(idx_hbm.at[wid], idx_v)
    gathers = [
        pltpu.async_copy(
            table_hbm.at[idx_v.at[j]],
            rows_v.at[pl.ds(j * _CHUNK, _CHUNK)],
            gather_sems[j],
        )
        for j in range(_N_CHUNKS)
    ]
    outs = []
    for j in range(_N_CHUNKS):
      gathers[j].wait()
      outs.append(
          pltpu.async_copy(
              rows_v.at[pl.ds(j * _CHUNK, _CHUNK)],
              out_hbm.at[pl.ds(base + j * _CHUNK, _CHUNK)],
              out_sem,
          )
      )
    for c in outs:
      c.wait()

  return k(idx, table)


def kernel(user_ids, user_embedding):
  idx = user_ids.astype(jnp.int32).reshape(_NW, _N_CHUNKS, _CHUNK)
  return _gather_call(idx, user_embedding)


# speedup vs baseline: 1.0088x; 1.0088x over previous
import functools

import jax
import jax.numpy as jnp
from jax import lax
from jax.experimental import pallas as pl
from jax.experimental.pallas import tpu as pltpu
from jax.experimental.pallas import tpu_sc as plsc

NUM_USERS = 100000
EMBED_DIM = 128
BATCH = 16384

_NC = 2
_NS = 16
_NW = _NC * _NS
_B_PER_W = BATCH // _NW
_CHUNK = 128
_N_CHUNKS = _B_PER_W // _CHUNK


def _gather_call(idx, table):
  mesh = plsc.VectorSubcoreMesh(core_axis_name="c", subcore_axis_name="s")

  @functools.partial(
      pl.kernel,
      mesh=mesh,
      out_type=jax.ShapeDtypeStruct((BATCH, EMBED_DIM), jnp.float32),
      scratch_types=[
          pltpu.VMEM((_N_CHUNKS, _CHUNK), jnp.int32),
          pltpu.VMEM((_B_PER_W, EMBED_DIM), jnp.float32),
          *([pltpu.SemaphoreType.DMA] * _N_CHUNKS),
          pltpu.SemaphoreType.DMA,
      ],
  )
  def k(idx_hbm, table_hbm, out_hbm, idx_v, rows_v, *sems):
    gather_sems, out_sem = sems[:_N_CHUNKS], sems[_N_CHUNKS]
    wid = lax.axis_index("s") * _NC + lax.axis_index("c")
    base = wid * _B_PER_W
    pltpu.sync_copy(idx_hbm.at[wid], idx_v)
    gathers = [
        pltpu.async_copy(
            table_hbm.at[idx_v.at[j]],
            rows_v.at[pl.ds(j * _CHUNK, _CHUNK)],
            gather_sems[j],
        )
        for j in range(_N_CHUNKS)
    ]
    outs = []
    for j in range(_N_CHUNKS):
      gathers[j].wait()
      outs.append(
          pltpu.async_copy(
              rows_v.at[pl.ds(j * _CHUNK, _CHUNK)],
              out_hbm.at[pl.ds(base + j * _CHUNK, _CHUNK)],
              out_sem,
          )
      )
    for c in outs:
      c.wait()

  return k(idx, table)


def kernel(user_ids, user_embedding):
  idx = user_ids.astype(jnp.int32).reshape(_NW, _N_CHUNKS, _CHUNK)
  return _gather_call(idx, user_embedding)


# speedup vs baseline: 1.0128x; 1.0040x over previous
import functools

import jax
import jax.numpy as jnp
from jax import lax
from jax.experimental import pallas as pl
from jax.experimental.pallas import tpu as pltpu
from jax.experimental.pallas import tpu_sc as plsc

NUM_USERS = 100000
EMBED_DIM = 128
BATCH = 16384

_NC = 2
_NS = 16
_NW = _NC * _NS
_B_PER_W = BATCH // _NW
_CHUNK = 128
_N_CHUNKS = _B_PER_W // _CHUNK


def _gather_call(idx, table):
  mesh = plsc.VectorSubcoreMesh(core_axis_name="c", subcore_axis_name="s")

  @functools.partial(
      pl.kernel,
      mesh=mesh,
      out_type=jax.ShapeDtypeStruct((BATCH, EMBED_DIM), jnp.float32),
      scratch_types=[
          pltpu.VMEM((_N_CHUNKS, _CHUNK), jnp.int32),
          pltpu.VMEM((_B_PER_W, EMBED_DIM), jnp.float32),
          *([pltpu.SemaphoreType.DMA] * _N_CHUNKS),
          pltpu.SemaphoreType.DMA,
      ],
  )
  def k(idx_hbm, table_hbm, out_hbm, idx_v, rows_v, *sems):
    gather_sems, out_sem = sems[:_N_CHUNKS], sems[_N_CHUNKS]
    wid = lax.axis_index("s") * _NC + lax.axis_index("c")
    base = wid * _B_PER_W
    idx_copies = [
        pltpu.async_copy(idx_hbm.at[wid].at[j], idx_v.at[j], gather_sems[j])
        for j in range(_N_CHUNKS)
    ]
    gathers = []
    for j in range(_N_CHUNKS):
      idx_copies[j].wait()
      gathers.append(
          pltpu.async_copy(
              table_hbm.at[idx_v.at[j]],
              rows_v.at[pl.ds(j * _CHUNK, _CHUNK)],
              gather_sems[j],
          )
      )
    outs = []
    for j in range(_N_CHUNKS):
      gathers[j].wait()
      outs.append(
          pltpu.async_copy(
              rows_v.at[pl.ds(j * _CHUNK, _CHUNK)],
              out_hbm.at[pl.ds(base + j * _CHUNK, _CHUNK)],
              out_sem,
          )
      )
    for c in outs:
      c.wait()

  return k(idx, table)


def kernel(user_ids, user_embedding):
  idx = user_ids.astype(jnp.int32).reshape(_NW, _N_CHUNKS, _CHUNK)
  return _gather_call(idx, user_embedding)
